# baseline (speedup 1.0000x reference)
import functools

import jax
import jax.numpy as jnp
from jax import lax
from jax.experimental import pallas as pl
from jax.experimental.pallas import tpu as pltpu
from jax.experimental.pallas import tpu_sc as plsc

ROWS = 64
N = 8192
K_OUT = 64
INT_MIN = -2147483648
U = 4


def _sort16_desc(v):
    k, _ = plsc.sort_key_val(v, v, descending=True)
    return k


def _rev(v):
    return lax.rev(v, (0,))


def _sort64_desc(v0, v1, v2, v3):
    a0, a1, a2, a3 = (_sort16_desc(v) for v in (v0, v1, v2, v3))

    def merge2(a, b):
        rb = _rev(b)
        return _sort16_desc(jnp.maximum(a, rb)), _sort16_desc(jnp.minimum(a, rb))

    a0, a1 = merge2(a0, a1)
    b0, b1 = merge2(a2, a3)
    rb0, rb1 = _rev(b1), _rev(b0)
    hi0, hi1 = jnp.maximum(a0, rb0), jnp.maximum(a1, rb1)
    lo0, lo1 = jnp.minimum(a0, rb0), jnp.minimum(a1, rb1)
    u0, u1 = jnp.maximum(hi0, hi1), jnp.minimum(hi0, hi1)
    u2, u3 = jnp.maximum(lo0, lo1), jnp.minimum(lo0, lo1)
    return (_sort16_desc(u0), _sort16_desc(u1),
            _sort16_desc(u2), _sort16_desc(u3))


def _to_key(x16):
    b = lax.bitcast_convert_type(x16, jnp.int32)
    return jnp.where(b < 0, ~b, b | jnp.int32(INT_MIN))


def _pcnt(mask):
    return plsc.all_reduce_population_count(mask)[0]


def _sc_topk_body(x_hbm, out_hbm, xv, keys, cbuf, outv, sem):
    del sem
    wid = lax.axis_index("s") * 2 + lax.axis_index("c")
    lane = lax.broadcasted_iota(jnp.int32, (16,), 0)

    def do_row(j, _):
        row = wid * 2 + j
        pltpu.sync_copy(x_hbm.at[row], xv)

        m31 = jnp.int32(INT_MIN)

        def part31(i, carry):
            o1, o0 = carry
            for u in range(U):
                x16 = xv[pl.ds(i * (16 * U) + u * 16, 16)]
                k16 = _to_key(x16)
                bit = (k16 & m31) != 0
                nbit = jnp.logical_not(bit)
                plsc.store_compressed(keys.at[pl.ds(o1, 16)], k16, mask=bit)
                plsc.store_compressed(keys.at[pl.ds(o0, 16)], k16, mask=nbit)
                pc1 = _pcnt(bit)
                o1 = o1 + pc1
                o0 = o0 + (16 - pc1)
            return o1, o0

        o1, o0 = lax.fori_loop(0, N // (16 * U), part31,
                               (jnp.int32(0), jnp.int32(N)))
        cnt1 = o1
        r_left = jnp.int32(K_OUT)
        take1 = cnt1 >= r_left
        pfx = jnp.where(take1, m31, jnp.int32(0))
        r_left = jnp.where(take1, r_left, r_left - cnt1)
        n_cur = jnp.where(take1, cnt1, jnp.int32(N) - cnt1)
        src = jnp.where(take1, jnp.int32(0), jnp.int32(N))
        d1 = jnp.int32(2 * N)
        d0 = jnp.where(take1, jnp.int32(N), jnp.int32(0))

        for t in range(30, -1, -1):
            m = jnp.int32(1 << t)
            nch = lax.shift_right_arithmetic(n_cur + (16 * U - 1), 6)

            def part(i, carry, src=src, m=m, n_cur=n_cur):
                o1, o0 = carry
                for u in range(U):
                    base = i * (16 * U) + u * 16
                    k16 = keys[pl.ds(src + base, 16)]
                    rem = n_cur - base
                    valid = lane < rem
                    bit = (k16 & m) != 0
                    m1 = jnp.logical_and(bit, valid)
                    m0 = jnp.logical_and(jnp.logical_not(bit), valid)
                    plsc.store_compressed(keys.at[pl.ds(o1, 16)], k16, mask=m1)
                    plsc.store_compressed(keys.at[pl.ds(o0, 16)], k16, mask=m0)
                    pc1 = _pcnt(m1)
                    pcv = _pcnt(valid)
                    o1 = o1 + pc1
                    o0 = o0 + (pcv - pc1)
                return o1, o0

            o1, o0 = lax.fori_loop(0, nch, part, (d1, d0))
            cnt1 = o1 - d1
            take1 = cnt1 >= r_left
            pfx = jnp.where(take1, pfx | m, pfx)
            r_left = jnp.where(take1, r_left, r_left - cnt1)
            n_new = jnp.where(take1, cnt1, n_cur - cnt1)
            src_new = jnp.where(take1, d1, d0)
            d0_new = jnp.where(take1, d0, d1)
            d1, d0, src, n_cur = src, d0_new, src_new, n_new

        pb = jnp.where(pfx < 0, pfx & jnp.int32(0x7FFFFFFF), ~pfx)
        v64 = lax.bitcast_convert_type(jnp.full((16,), pb, jnp.int32), jnp.float32)

        for q in range(K_OUT // 16):
            cbuf[pl.ds(q * 16, 16)] = v64

        def compact(i, cur):
            for u in range(U):
                x16 = xv[pl.ds(i * (16 * U) + u * 16, 16)]
                msk = x16 > v64
                plsc.store_compressed(cbuf.at[pl.ds(cur, 16)], x16, mask=msk)
                cur = cur + _pcnt(msk)
            return cur

        lax.fori_loop(0, N // (16 * U), compact, jnp.int32(0))

        s0, s1, s2, s3 = _sort64_desc(
            cbuf[pl.ds(0, 16)], cbuf[pl.ds(16, 16)],
            cbuf[pl.ds(32, 16)], cbuf[pl.ds(48, 16)])
        outv[pl.ds(0, 16)] = s0
        outv[pl.ds(16, 16)] = s1
        outv[pl.ds(32, 16)] = s2
        outv[pl.ds(48, 16)] = s3
        pltpu.sync_copy(outv, out_hbm.at[row])
        return _

    lax.fori_loop(0, 2, do_row, 0)


@jax.jit
def kernel(inputs):
    mesh = plsc.VectorSubcoreMesh(core_axis_name="c", subcore_axis_name="s")
    f = functools.partial(
        pl.kernel,
        mesh=mesh,
        compiler_params=pltpu.CompilerParams(needs_layout_passes=False),
        out_type=jax.ShapeDtypeStruct((ROWS, K_OUT), jnp.float32),
        scratch_types=[
            pltpu.VMEM((N,), jnp.float32),
            pltpu.VMEM((3 * N + 64,), jnp.int32),
            pltpu.VMEM((K_OUT + 16,), jnp.float32),
            pltpu.VMEM((K_OUT,), jnp.float32),
            pltpu.SemaphoreType.DMA,
        ],
    )(_sc_topk_body)
    return f(inputs)


# speedup vs baseline: 1.2849x; 1.2849x over previous
import functools

import jax
import jax.numpy as jnp
from jax import lax
from jax.experimental import pallas as pl
from jax.experimental.pallas import tpu as pltpu
from jax.experimental.pallas import tpu_sc as plsc

ROWS = 64
N = 8192
K_OUT = 64
INT_MIN = -2147483648
U = 4


def _sort16_desc(v):
    k, _ = plsc.sort_key_val(v, v, descending=True)
    return k


def _rev(v):
    return lax.rev(v, (0,))


def _sort64_desc(v0, v1, v2, v3):
    a0, a1, a2, a3 = (_sort16_desc(v) for v in (v0, v1, v2, v3))

    def merge2(a, b):
        rb = _rev(b)
        return _sort16_desc(jnp.maximum(a, rb)), _sort16_desc(jnp.minimum(a, rb))

    a0, a1 = merge2(a0, a1)
    b0, b1 = merge2(a2, a3)
    rb0, rb1 = _rev(b1), _rev(b0)
    hi0, hi1 = jnp.maximum(a0, rb0), jnp.maximum(a1, rb1)
    lo0, lo1 = jnp.minimum(a0, rb0), jnp.minimum(a1, rb1)
    u0, u1 = jnp.maximum(hi0, hi1), jnp.minimum(hi0, hi1)
    u2, u3 = jnp.maximum(lo0, lo1), jnp.minimum(lo0, lo1)
    return (_sort16_desc(u0), _sort16_desc(u1),
            _sort16_desc(u2), _sort16_desc(u3))


def _to_key(x16):
    b = lax.bitcast_convert_type(x16, jnp.int32)
    return jnp.where(b < 0, ~b, b | jnp.int32(INT_MIN))


def _pcnt(mask):
    return plsc.all_reduce_population_count(mask)[0]


def _sc_topk_body(x_hbm, out_hbm, xv, keys, cbuf, outv, sem):
    del sem
    wid = lax.axis_index("s") * 2 + lax.axis_index("c")
    lane = lax.broadcasted_iota(jnp.int32, (16,), 0)

    def do_row(j, _):
        row = wid * 2 + j
        pltpu.sync_copy(x_hbm.at[row], xv)

        m31 = jnp.int32(INT_MIN)

        def part31(i, carry):
            o1, o0 = carry
            for u in range(U):
                x16 = xv[pl.ds(i * (16 * U) + u * 16, 16)]
                k16 = _to_key(x16)
                bit = (k16 & m31) != 0
                nbit = jnp.logical_not(bit)
                plsc.store_compressed(keys.at[pl.ds(o1, 16)], k16, mask=bit)
                plsc.store_compressed(keys.at[pl.ds(o0, 16)], k16, mask=nbit)
                pc1 = _pcnt(bit)
                o1 = o1 + pc1
                o0 = o0 + (16 - pc1)
            return o1, o0

        o1, o0 = lax.fori_loop(0, N // (16 * U), part31,
                               (jnp.int32(0), jnp.int32(N)))
        cnt1 = o1
        r_left = jnp.int32(K_OUT)
        take1 = cnt1 >= r_left
        pfx = jnp.where(take1, m31, jnp.int32(0))
        r_left = jnp.where(take1, r_left, r_left - cnt1)
        n_cur = jnp.where(take1, cnt1, jnp.int32(N) - cnt1)
        src = jnp.where(take1, jnp.int32(0), jnp.int32(N))
        d1 = jnp.int32(2 * N)
        d0 = jnp.where(take1, jnp.int32(N), jnp.int32(0))

        def bitstep(it, state):
            src, d1, d0, n_cur, r_left, pfx = state
            m = lax.shift_left(jnp.int32(1), 30 - it)
            nch = lax.shift_right_arithmetic(n_cur + (16 * U - 1), 6)

            def part(i, carry):
                o1, o0 = carry
                for u in range(U):
                    base = i * (16 * U) + u * 16
                    k16 = keys[pl.ds(src + base, 16)]
                    rem = n_cur - base
                    valid = lane < rem
                    bit = (k16 & m) != 0
                    m1 = jnp.logical_and(bit, valid)
                    m0 = jnp.logical_and(jnp.logical_not(bit), valid)
                    plsc.store_compressed(keys.at[pl.ds(o1, 16)], k16, mask=m1)
                    plsc.store_compressed(keys.at[pl.ds(o0, 16)], k16, mask=m0)
                    pc1 = _pcnt(m1)
                    pcv = _pcnt(valid)
                    o1 = o1 + pc1
                    o0 = o0 + (pcv - pc1)
                return o1, o0

            o1, o0 = lax.fori_loop(0, nch, part, (d1, d0))
            cnt1 = o1 - d1
            take1 = cnt1 >= r_left
            pfx = jnp.where(take1, pfx | m, pfx)
            r_left = jnp.where(take1, r_left, r_left - cnt1)
            n_new = jnp.where(take1, cnt1, n_cur - cnt1)
            src_new = jnp.where(take1, d1, d0)
            d0_new = jnp.where(take1, d0, d1)
            return (src_new, src, d0_new, n_new, r_left, pfx)

        src, d1, d0, n_cur, r_left, pfx = lax.fori_loop(
            0, 31, bitstep, (src, d1, d0, n_cur, r_left, pfx))

        pb = jnp.where(pfx < 0, pfx & jnp.int32(0x7FFFFFFF), ~pfx)
        v64 = lax.bitcast_convert_type(jnp.full((16,), pb, jnp.int32), jnp.float32)

        for q in range(K_OUT // 16):
            cbuf[pl.ds(q * 16, 16)] = v64

        def compact(i, cur):
            for u in range(U):
                x16 = xv[pl.ds(i * (16 * U) + u * 16, 16)]
                msk = x16 > v64
                plsc.store_compressed(cbuf.at[pl.ds(cur, 16)], x16, mask=msk)
                cur = cur + _pcnt(msk)
            return cur

        lax.fori_loop(0, N // (16 * U), compact, jnp.int32(0))

        s0, s1, s2, s3 = _sort64_desc(
            cbuf[pl.ds(0, 16)], cbuf[pl.ds(16, 16)],
            cbuf[pl.ds(32, 16)], cbuf[pl.ds(48, 16)])
        outv[pl.ds(0, 16)] = s0
        outv[pl.ds(16, 16)] = s1
        outv[pl.ds(32, 16)] = s2
        outv[pl.ds(48, 16)] = s3
        pltpu.sync_copy(outv, out_hbm.at[row])
        return _

    lax.fori_loop(0, 2, do_row, 0)


@jax.jit
def kernel(inputs):
    mesh = plsc.VectorSubcoreMesh(core_axis_name="c", subcore_axis_name="s")
    f = functools.partial(
        pl.kernel,
        mesh=mesh,
        compiler_params=pltpu.CompilerParams(needs_layout_passes=False),
        out_type=jax.ShapeDtypeStruct((ROWS, K_OUT), jnp.float32),
        scratch_types=[
            pltpu.VMEM((N,), jnp.float32),
            pltpu.VMEM((3 * N + 64,), jnp.int32),
            pltpu.VMEM((K_OUT + 16,), jnp.float32),
            pltpu.VMEM((K_OUT,), jnp.float32),
            pltpu.SemaphoreType.DMA,
        ],
    )(_sc_topk_body)
    return f(inputs)


# speedup vs baseline: 1.2960x; 1.0086x over previous
import functools

import jax
import jax.numpy as jnp
from jax import lax
from jax.experimental import pallas as pl
from jax.experimental.pallas import tpu as pltpu
from jax.experimental.pallas import tpu_sc as plsc

ROWS = 64
N = 8192
K_OUT = 64
INT_MIN = -2147483648
U = 4
NEG_INF = float("-inf")


def _s16(v):
    k, _ = plsc.sort_key_val(v, v, descending=True)
    return k


def _rev(v):
    return lax.rev(v, (0,))


def _merge2(a, b):
    rb = _rev(b)
    return _s16(jnp.maximum(a, rb)), _s16(jnp.minimum(a, rb))


def _merge4(a0, a1, b0, b1):
    rb0, rb1 = _rev(b1), _rev(b0)
    hi0, hi1 = jnp.maximum(a0, rb0), jnp.maximum(a1, rb1)
    lo0, lo1 = jnp.minimum(a0, rb0), jnp.minimum(a1, rb1)
    u0, u1 = jnp.maximum(hi0, hi1), jnp.minimum(hi0, hi1)
    u2, u3 = jnp.maximum(lo0, lo1), jnp.minimum(lo0, lo1)
    return _s16(u0), _s16(u1), _s16(u2), _s16(u3)


def _top64_of_two64(a, b):
    a0, a1, a2, a3 = a
    b0, b1, b2, b3 = b
    c0 = jnp.maximum(a0, _rev(b3))
    c1 = jnp.maximum(a1, _rev(b2))
    c2 = jnp.maximum(a2, _rev(b1))
    c3 = jnp.maximum(a3, _rev(b0))
    p0, p2 = jnp.maximum(c0, c2), jnp.minimum(c0, c2)
    p1, p3 = jnp.maximum(c1, c3), jnp.minimum(c1, c3)
    q0, q1 = jnp.maximum(p0, p1), jnp.minimum(p0, p1)
    q2, q3 = jnp.maximum(p2, p3), jnp.minimum(p2, p3)
    return _s16(q0), _s16(q1), _s16(q2), _s16(q3)


def _sort64(v0, v1, v2, v3):
    a0, a1 = _merge2(_s16(v0), _s16(v1))
    b0, b1 = _merge2(_s16(v2), _s16(v3))
    return _merge4(a0, a1, b0, b1)


def _to_key(x16):
    b = lax.bitcast_convert_type(x16, jnp.int32)
    return jnp.where(b < 0, ~b, b | jnp.int32(INT_MIN))


def _from_key(k16):
    b = jnp.where(k16 < 0, k16 & jnp.int32(0x7FFFFFFF), ~k16)
    return lax.bitcast_convert_type(b, jnp.float32)


def _pcnt(mask):
    return plsc.all_reduce_population_count(mask)[0]


def _sc_topk_body(x_hbm, out_hbm, xv, keys, cbuf, outv, sem):
    del sem
    wid = lax.axis_index("s") * 2 + lax.axis_index("c")
    lane = lax.broadcasted_iota(jnp.int32, (16,), 0)

    def do_row(j, _):
        row = wid * 2 + j
        pltpu.sync_copy(x_hbm.at[row], xv)

        m31 = jnp.int32(INT_MIN)

        def part31(i, carry):
            o1, o0 = carry
            for u in range(U):
                x16 = xv[pl.ds(i * (16 * U) + u * 16, 16)]
                k16 = _to_key(x16)
                bit = (k16 & m31) != 0
                nbit = jnp.logical_not(bit)
                plsc.store_compressed(keys.at[pl.ds(o1, 16)], k16, mask=bit)
                plsc.store_compressed(keys.at[pl.ds(o0, 16)], k16, mask=nbit)
                pc1 = _pcnt(bit)
                o1 = o1 + pc1
                o0 = o0 + (16 - pc1)
            return o1, o0

        o1, o0 = lax.fori_loop(0, N // (16 * U), part31,
                               (jnp.int32(0), jnp.int32(N)))
        cnt1 = o1
        r_left = jnp.int32(K_OUT)
        take1 = cnt1 >= r_left
        pfx = jnp.where(take1, m31, jnp.int32(0))
        r_left = jnp.where(take1, r_left, r_left - cnt1)
        n_cur = jnp.where(take1, cnt1, jnp.int32(N) - cnt1)
        src = jnp.where(take1, jnp.int32(0), jnp.int32(N))
        d1 = jnp.int32(2 * N)
        d0 = jnp.where(take1, jnp.int32(N), jnp.int32(0))
        low = jnp.int32(0x7FFFFFFF)

        def cond(state):
            _, _, _, n_cur, _, _, _, it = state
            return jnp.logical_and(n_cur > K_OUT, it < 31)

        def bitstep(state):
            src, d1, d0, n_cur, r_left, pfx, low, it = state
            m = lax.shift_left(jnp.int32(1), 30 - it)
            nch = lax.shift_right_arithmetic(n_cur + (16 * U - 1), 6)

            def part(i, carry):
                o1, o0 = carry
                for u in range(U):
                    base = i * (16 * U) + u * 16
                    k16 = keys[pl.ds(src + base, 16)]
                    rem = n_cur - base
                    valid = lane < rem
                    bit = (k16 & m) != 0
                    m1 = jnp.logical_and(bit, valid)
                    m0 = jnp.logical_and(jnp.logical_not(bit), valid)
                    plsc.store_compressed(keys.at[pl.ds(o1, 16)], k16, mask=m1)
                    plsc.store_compressed(keys.at[pl.ds(o0, 16)], k16, mask=m0)
                    pc1 = _pcnt(m1)
                    pcv = _pcnt(valid)
                    o1 = o1 + pc1
                    o0 = o0 + (pcv - pc1)
                return o1, o0

            o1, o0 = lax.fori_loop(0, nch, part, (d1, d0))
            cnt1 = o1 - d1
            take1 = cnt1 >= r_left
            pfx = jnp.where(take1, pfx | m, pfx)
            r_left = jnp.where(take1, r_left, r_left - cnt1)
            n_new = jnp.where(take1, cnt1, n_cur - cnt1)
            src_new = jnp.where(take1, d1, d0)
            d0_new = jnp.where(take1, d0, d1)
            return (src_new, src, d0_new, n_new, r_left, pfx,
                    lax.shift_right_arithmetic(low, 1), it + 1)

        src, d1, d0, n_cur, r_left, pfx, low, _ = lax.while_loop(
            cond, bitstep, (src, d1, d0, n_cur, r_left, pfx, low, jnp.int32(0)))

        hi_key = (pfx | low) ^ jnp.int32(INT_MIN)
        hi_vec = jnp.full((16,), hi_key, jnp.int32)
        sgn = jnp.full((16,), INT_MIN, jnp.int32)
        ninf = jnp.full((16,), NEG_INF, jnp.float32)

        for q in range(8):
            cbuf[pl.ds(q * 16, 16)] = ninf

        def compact(i, cur):
            for u in range(U):
                x16 = xv[pl.ds(i * (16 * U) + u * 16, 16)]
                k16 = _to_key(x16)
                msk = (k16 ^ sgn) > hi_vec
                plsc.store_compressed(cbuf.at[pl.ds(cur, 16)], x16, mask=msk)
                cur = cur + _pcnt(msk)
            return cur

        cur = lax.fori_loop(0, N // (16 * U), compact, jnp.int32(0))

        for q in range(4):
            k16 = keys[pl.ds(src + q * 16, 16)]
            msk = lane + q * 16 < n_cur
            plsc.store_compressed(cbuf.at[pl.ds(cur + q * 16, 16)],
                                  _from_key(k16), mask=msk)

        a = _sort64(cbuf[pl.ds(0, 16)], cbuf[pl.ds(16, 16)],
                    cbuf[pl.ds(32, 16)], cbuf[pl.ds(48, 16)])
        b = _sort64(cbuf[pl.ds(64, 16)], cbuf[pl.ds(80, 16)],
                    cbuf[pl.ds(96, 16)], cbuf[pl.ds(112, 16)])
        s0, s1, s2, s3 = _top64_of_two64(a, b)
        outv[pl.ds(0, 16)] = s0
        outv[pl.ds(16, 16)] = s1
        outv[pl.ds(32, 16)] = s2
        outv[pl.ds(48, 16)] = s3
        pltpu.sync_copy(outv, out_hbm.at[row])
        return _

    lax.fori_loop(0, 2, do_row, 0)


@jax.jit
def kernel(inputs):
    mesh = plsc.VectorSubcoreMesh(core_axis_name="c", subcore_axis_name="s")
    f = functools.partial(
        pl.kernel,
        mesh=mesh,
        compiler_params=pltpu.CompilerParams(needs_layout_passes=False),
        out_type=jax.ShapeDtypeStruct((ROWS, K_OUT), jnp.float32),
        scratch_types=[
            pltpu.VMEM((N,), jnp.float32),
            pltpu.VMEM((3 * N + 64,), jnp.int32),
            pltpu.VMEM((128 + 80,), jnp.float32),
            pltpu.VMEM((K_OUT,), jnp.float32),
            pltpu.SemaphoreType.DMA,
        ],
    )(_sc_topk_body)
    return f(inputs)


# speedup vs baseline: 1.5868x; 1.2244x over previous
import functools

import jax
import jax.numpy as jnp
from jax import lax
from jax.experimental import pallas as pl
from jax.experimental.pallas import tpu as pltpu
from jax.experimental.pallas import tpu_sc as plsc

ROWS = 64
N = 8192
K_OUT = 64
INT_MIN = -2147483648
U = 4
NEG_INF = float("-inf")


def _s16(v):
    k, _ = plsc.sort_key_val(v, v, descending=True)
    return k


def _rev(v):
    return lax.rev(v, (0,))


def _merge2(a, b):
    rb = _rev(b)
    return _s16(jnp.maximum(a, rb)), _s16(jnp.minimum(a, rb))


def _merge4(a0, a1, b0, b1):
    rb0, rb1 = _rev(b1), _rev(b0)
    hi0, hi1 = jnp.maximum(a0, rb0), jnp.maximum(a1, rb1)
    lo0, lo1 = jnp.minimum(a0, rb0), jnp.minimum(a1, rb1)
    u0, u1 = jnp.maximum(hi0, hi1), jnp.minimum(hi0, hi1)
    u2, u3 = jnp.maximum(lo0, lo1), jnp.minimum(lo0, lo1)
    return _s16(u0), _s16(u1), _s16(u2), _s16(u3)


def _top64_of_two64(a, b):
    a0, a1, a2, a3 = a
    b0, b1, b2, b3 = b
    c0 = jnp.maximum(a0, _rev(b3))
    c1 = jnp.maximum(a1, _rev(b2))
    c2 = jnp.maximum(a2, _rev(b1))
    c3 = jnp.maximum(a3, _rev(b0))
    p0, p2 = jnp.maximum(c0, c2), jnp.minimum(c0, c2)
    p1, p3 = jnp.maximum(c1, c3), jnp.minimum(c1, c3)
    q0, q1 = jnp.maximum(p0, p1), jnp.minimum(p0, p1)
    q2, q3 = jnp.maximum(p2, p3), jnp.minimum(p2, p3)
    return _s16(q0), _s16(q1), _s16(q2), _s16(q3)


def _sort64(v0, v1, v2, v3):
    a0, a1 = _merge2(_s16(v0), _s16(v1))
    b0, b1 = _merge2(_s16(v2), _s16(v3))
    return _merge4(a0, a1, b0, b1)


def _to_key(x16):
    b = lax.bitcast_convert_type(x16, jnp.int32)
    return jnp.where(b < 0, ~b, b | jnp.int32(INT_MIN))


def _from_key(k16):
    b = jnp.where(k16 < 0, k16 & jnp.int32(0x7FFFFFFF), ~k16)
    return lax.bitcast_convert_type(b, jnp.float32)


def _pcnt(mask):
    return plsc.all_reduce_population_count(mask)[0]


def _sc_topk_body(x_hbm, out_hbm, xv, keys, cbuf, outv, sem):
    del sem
    wid = lax.axis_index("s") * 2 + lax.axis_index("c")
    lane = lax.broadcasted_iota(jnp.int32, (16,), 0)

    def do_row(j, _):
        row = wid * 2 + j
        pltpu.sync_copy(x_hbm.at[row], xv)

        ninf = jnp.full((16,), NEG_INF, jnp.float32)
        for q in range(8):
            cbuf[pl.ds(q * 16, 16)] = ninf

        def harvest(hi_base, hi_cnt, cur):
            def cp(i, cur):
                k16 = keys[pl.ds(hi_base + i * 16, 16)]
                msk = lane + i * 16 < hi_cnt
                plsc.store_compressed(cbuf.at[pl.ds(cur, 16)],
                                      _from_key(k16), mask=msk)
                return cur + jnp.minimum(hi_cnt - i * 16, jnp.int32(16))

            ncp = lax.shift_right_arithmetic(hi_cnt + 15, 4)
            return lax.fori_loop(0, ncp, cp, cur)

        m31 = jnp.int32(INT_MIN)

        def part31(i, carry):
            o1, o0 = carry
            for u in range(U):
                x16 = xv[pl.ds(i * (16 * U) + u * 16, 16)]
                k16 = _to_key(x16)
                bit = (k16 & m31) != 0
                nbit = jnp.logical_not(bit)
                plsc.store_compressed(keys.at[pl.ds(o1, 16)], k16, mask=bit)
                plsc.store_compressed(keys.at[pl.ds(o0, 16)], k16, mask=nbit)
                pc1 = _pcnt(bit)
                o1 = o1 + pc1
                o0 = o0 + (16 - pc1)
            return o1, o0

        o1, o0 = lax.fori_loop(0, N // (16 * U), part31,
                               (jnp.int32(0), jnp.int32(N)))
        cnt1 = o1
        r_left = jnp.int32(K_OUT)
        take1 = cnt1 >= r_left
        cur = harvest(jnp.int32(0), jnp.where(take1, 0, cnt1), jnp.int32(0))
        r_left = jnp.where(take1, r_left, r_left - cnt1)
        n_cur = jnp.where(take1, cnt1, jnp.int32(N) - cnt1)
        src = jnp.where(take1, jnp.int32(0), jnp.int32(N))
        d1 = jnp.int32(2 * N)
        d0 = jnp.where(take1, jnp.int32(N), jnp.int32(0))

        def cond(state):
            _, _, _, n_cur, _, _, it = state
            return jnp.logical_and(n_cur > K_OUT, it < 31)

        def bitstep(state):
            src, d1, d0, n_cur, r_left, cur, it = state
            m = lax.shift_left(jnp.int32(1), 30 - it)
            nch = lax.shift_right_arithmetic(n_cur + (16 * U - 1), 6)

            def part(i, carry):
                o1, o0 = carry
                for u in range(U):
                    base = i * (16 * U) + u * 16
                    k16 = keys[pl.ds(src + base, 16)]
                    rem = n_cur - base
                    valid = lane < rem
                    bit = (k16 & m) != 0
                    m1 = jnp.logical_and(bit, valid)
                    m0 = jnp.logical_and(jnp.logical_not(bit), valid)
                    plsc.store_compressed(keys.at[pl.ds(o1, 16)], k16, mask=m1)
                    plsc.store_compressed(keys.at[pl.ds(o0, 16)], k16, mask=m0)
                    pc1 = _pcnt(m1)
                    pcv = jnp.clip(rem, 0, 16)
                    o1 = o1 + pc1
                    o0 = o0 + (pcv - pc1)
                return o1, o0

            o1, o0 = lax.fori_loop(0, nch, part, (d1, d0))
            cnt1 = o1 - d1
            take1 = cnt1 >= r_left
            cur = harvest(d1, jnp.where(take1, 0, cnt1), cur)
            r_left = jnp.where(take1, r_left, r_left - cnt1)
            n_new = jnp.where(take1, cnt1, n_cur - cnt1)
            src_new = jnp.where(take1, d1, d0)
            d0_new = jnp.where(take1, d0, d1)
            return (src_new, src, d0_new, n_new, r_left, cur, it + 1)

        src, d1, d0, n_cur, r_left, cur, _ = lax.while_loop(
            cond, bitstep,
            (src, d1, d0, n_cur, r_left, cur, jnp.int32(0)))

        for q in range(4):
            k16 = keys[pl.ds(src + q * 16, 16)]
            msk = lane + q * 16 < n_cur
            plsc.store_compressed(cbuf.at[pl.ds(cur + q * 16, 16)],
                                  _from_key(k16), mask=msk)

        a = _sort64(cbuf[pl.ds(0, 16)], cbuf[pl.ds(16, 16)],
                    cbuf[pl.ds(32, 16)], cbuf[pl.ds(48, 16)])
        b = _sort64(cbuf[pl.ds(64, 16)], cbuf[pl.ds(80, 16)],
                    cbuf[pl.ds(96, 16)], cbuf[pl.ds(112, 16)])
        s0, s1, s2, s3 = _top64_of_two64(a, b)
        outv[pl.ds(0, 16)] = s0
        outv[pl.ds(16, 16)] = s1
        outv[pl.ds(32, 16)] = s2
        outv[pl.ds(48, 16)] = s3
        pltpu.sync_copy(outv, out_hbm.at[row])
        return _

    lax.fori_loop(0, 2, do_row, 0)


@jax.jit
def kernel(inputs):
    mesh = plsc.VectorSubcoreMesh(core_axis_name="c", subcore_axis_name="s")
    f = functools.partial(
        pl.kernel,
        mesh=mesh,
        compiler_params=pltpu.CompilerParams(needs_layout_passes=False),
        out_type=jax.ShapeDtypeStruct((ROWS, K_OUT), jnp.float32),
        scratch_types=[
            pltpu.VMEM((N,), jnp.float32),
            pltpu.VMEM((3 * N + 64,), jnp.int32),
            pltpu.VMEM((128 + 80,), jnp.float32),
            pltpu.VMEM((K_OUT,), jnp.float32),
            pltpu.SemaphoreType.DMA,
        ],
    )(_sc_topk_body)
    return f(inputs)


# speedup vs baseline: 1.5902x; 1.0021x over previous
import functools

import jax
import jax.numpy as jnp
from jax import lax
from jax.experimental import pallas as pl
from jax.experimental.pallas import tpu as pltpu
from jax.experimental.pallas import tpu_sc as plsc

ROWS = 64
N = 8192
K_OUT = 64
INT_MIN = -2147483648
U = 4
NEG_INF = float("-inf")


def _s16(v):
    k, _ = plsc.sort_key_val(v, v, descending=True)
    return k


def _rev(v):
    return lax.rev(v, (0,))


def _merge2(a, b):
    rb = _rev(b)
    return _s16(jnp.maximum(a, rb)), _s16(jnp.minimum(a, rb))


def _merge4(a0, a1, b0, b1):
    rb0, rb1 = _rev(b1), _rev(b0)
    hi0, hi1 = jnp.maximum(a0, rb0), jnp.maximum(a1, rb1)
    lo0, lo1 = jnp.minimum(a0, rb0), jnp.minimum(a1, rb1)
    u0, u1 = jnp.maximum(hi0, hi1), jnp.minimum(hi0, hi1)
    u2, u3 = jnp.maximum(lo0, lo1), jnp.minimum(lo0, lo1)
    return _s16(u0), _s16(u1), _s16(u2), _s16(u3)


def _top64_of_two64(a, b):
    a0, a1, a2, a3 = a
    b0, b1, b2, b3 = b
    c0 = jnp.maximum(a0, _rev(b3))
    c1 = jnp.maximum(a1, _rev(b2))
    c2 = jnp.maximum(a2, _rev(b1))
    c3 = jnp.maximum(a3, _rev(b0))
    p0, p2 = jnp.maximum(c0, c2), jnp.minimum(c0, c2)
    p1, p3 = jnp.maximum(c1, c3), jnp.minimum(c1, c3)
    q0, q1 = jnp.maximum(p0, p1), jnp.minimum(p0, p1)
    q2, q3 = jnp.maximum(p2, p3), jnp.minimum(p2, p3)
    return _s16(q0), _s16(q1), _s16(q2), _s16(q3)


def _sort64(v0, v1, v2, v3):
    a0, a1 = _merge2(_s16(v0), _s16(v1))
    b0, b1 = _merge2(_s16(v2), _s16(v3))
    return _merge4(a0, a1, b0, b1)


def _to_key(x16):
    b = lax.bitcast_convert_type(x16, jnp.int32)
    return jnp.where(b < 0, ~b, b | jnp.int32(INT_MIN))


def _from_key(k16):
    b = jnp.where(k16 < 0, k16 & jnp.int32(0x7FFFFFFF), ~k16)
    return lax.bitcast_convert_type(b, jnp.float32)


def _pcnt(mask):
    return plsc.all_reduce_population_count(mask)[0]


def _sc_topk_body(x_hbm, out_hbm, xv, keys, cbuf, outv, sem):
    del sem
    wid = lax.axis_index("s") * 2 + lax.axis_index("c")
    lane = lax.broadcasted_iota(jnp.int32, (16,), 0)

    def do_row(j, _):
        row = wid * 2 + j
        pltpu.sync_copy(x_hbm.at[row], xv)

        ninf = jnp.full((16,), NEG_INF, jnp.float32)
        for q in range(8):
            cbuf[pl.ds(q * 16, 16)] = ninf

        def harvest(hi_base, hi_cnt, cur):
            def cp(i, cur):
                k16 = keys[pl.ds(hi_base + i * 16, 16)]
                msk = lane + i * 16 < hi_cnt
                plsc.store_compressed(cbuf.at[pl.ds(cur, 16)],
                                      _from_key(k16), mask=msk)
                return cur + jnp.minimum(hi_cnt - i * 16, jnp.int32(16))

            ncp = lax.shift_right_arithmetic(hi_cnt + 15, 4)
            return lax.fori_loop(0, ncp, cp, cur)

        m31 = jnp.int32(INT_MIN)

        def part31(i, carry):
            o1, o0 = carry
            for u in range(U):
                x16 = xv[pl.ds(i * (16 * U) + u * 16, 16)]
                k16 = _to_key(x16)
                bit = (k16 & m31) != 0
                nbit = jnp.logical_not(bit)
                plsc.store_compressed(keys.at[pl.ds(o1, 16)], k16, mask=bit)
                plsc.store_compressed(keys.at[pl.ds(o0, 16)], k16, mask=nbit)
                pc1 = _pcnt(bit)
                o1 = o1 + pc1
                o0 = o0 + (16 - pc1)
            return o1, o0

        o1, o0 = lax.fori_loop(0, N // (16 * U), part31,
                               (jnp.int32(0), jnp.int32(N)))
        cnt1 = o1
        r_left = jnp.int32(K_OUT)
        take1 = cnt1 >= r_left
        cur = harvest(jnp.int32(0), jnp.where(take1, 0, cnt1), jnp.int32(0))
        r_left = jnp.where(take1, r_left, r_left - cnt1)
        n_cur = jnp.where(take1, cnt1, jnp.int32(N) - cnt1)
        src = jnp.where(take1, jnp.int32(0), jnp.int32(N))
        d1 = jnp.int32(2 * N)
        d0 = jnp.where(take1, jnp.int32(N), jnp.int32(0))

        def cond(state):
            _, _, _, n_cur, _, _, it = state
            return jnp.logical_and(n_cur > K_OUT, it < 31)

        def bitstep(state):
            src, d1, d0, n_cur, r_left, cur, it = state
            m = lax.shift_left(jnp.int32(1), 30 - it)
            nch = lax.shift_right_arithmetic(n_cur + (16 * U - 1), 6)

            def part(i, carry):
                o1, o0 = carry
                for u in range(U):
                    base = i * (16 * U) + u * 16
                    k16 = keys[pl.ds(src + base, 16)]
                    rem = n_cur - base
                    valid = lane < rem
                    bit = (k16 & m) != 0
                    m1 = jnp.logical_and(bit, valid)
                    m0 = jnp.logical_and(jnp.logical_not(bit), valid)
                    plsc.store_compressed(keys.at[pl.ds(o1, 16)], k16, mask=m1)
                    plsc.store_compressed(keys.at[pl.ds(o0, 16)], k16, mask=m0)
                    pc1 = _pcnt(m1)
                    pcv = jnp.clip(rem, 0, 16)
                    o1 = o1 + pc1
                    o0 = o0 + (pcv - pc1)
                return o1, o0

            o1, o0 = lax.fori_loop(0, nch, part, (d1, d0))
            cnt1 = o1 - d1
            take1 = cnt1 >= r_left
            cur = harvest(d1, jnp.where(take1, 0, cnt1), cur)
            r_left = jnp.where(take1, r_left, r_left - cnt1)
            n_new = jnp.where(take1, cnt1, n_cur - cnt1)
            src_new = jnp.where(take1, d1, d0)
            d0_new = jnp.where(take1, d0, d1)
            return (src_new, src, d0_new, n_new, r_left, cur, it + 1)

        src, d1, d0, n_cur, r_left, cur, _ = lax.while_loop(
            cond, bitstep,
            (src, d1, d0, n_cur, r_left, cur, jnp.int32(0)))

        for q in range(4):
            k16 = keys[pl.ds(src + q * 16, 16)]
            msk = lane + q * 16 < n_cur
            plsc.store_compressed(cbuf.at[pl.ds(cur + q * 16, 16)],
                                  _from_key(k16), mask=msk)

        a = _sort64(cbuf[pl.ds(0, 16)], cbuf[pl.ds(16, 16)],
                    cbuf[pl.ds(32, 16)], cbuf[pl.ds(48, 16)])
        b = _sort64(cbuf[pl.ds(64, 16)], cbuf[pl.ds(80, 16)],
                    cbuf[pl.ds(96, 16)], cbuf[pl.ds(112, 16)])
        s0, s1, s2, s3 = _top64_of_two64(a, b)
        outv[pl.ds(0, 16)] = s0
        outv[pl.ds(16, 16)] = s1
        outv[pl.ds(32, 16)] = s2
        outv[pl.ds(48, 16)] = s3
        pltpu.sync_copy(outv, out_hbm.at[row])
        return _

    lax.fori_loop(0, 2, do_row, 0)


@jax.jit
def kernel(inputs):
    mesh = plsc.VectorSubcoreMesh(core_axis_name="c", subcore_axis_name="s")
    f = functools.partial(
        pl.kernel,
        mesh=mesh,
        compiler_params=pltpu.CompilerParams(
            needs_layout_passes=False,
            skip_device_barrier=True,
            disable_bounds_checks=True,
            disable_semaphore_checks=True,
        ),
        out_type=jax.ShapeDtypeStruct((ROWS, K_OUT), jnp.float32),
        scratch_types=[
            pltpu.VMEM((N,), jnp.float32),
            pltpu.VMEM((3 * N + 64,), jnp.int32),
            pltpu.VMEM((128 + 80,), jnp.float32),
            pltpu.VMEM((K_OUT,), jnp.float32),
            pltpu.SemaphoreType.DMA,
        ],
    )(_sc_topk_body)
    return f(inputs)
